# initial kernel scaffold (unmeasured)
import jax
import jax.numpy as jnp
from jax import lax
from jax.experimental import pallas as pl
from jax.experimental.pallas import tpu as pltpu

N_DEV = 32
M = 4096
N = 2048
CHUNK = M // N_DEV


def kernel(x, w_mat, scale_x, scale_w):
    def body(x_ref, w_ref, sx_ref, sw_ref, out_ref,
             comm_ref, send_sem, recv_sem, credit_sem):
        my = lax.axis_index("i")
        left = jnp.mod(my - 1, N_DEV)
        right = jnp.mod(my + 1, N_DEV)

        barrier = pltpu.get_barrier_semaphore()
        for nbr in (left, right):
            pl.semaphore_signal(
                barrier, inc=1,
                device_id=(nbr,), device_id_type=pl.DeviceIdType.MESH,
            )
        pl.semaphore_wait(barrier, 2)

        acc = lax.dot_general(
            x_ref[:, :], w_ref[:, :],
            dimension_numbers=(((1,), (0,)), ((), ())),
            preferred_element_type=jnp.int32,
        )
        out_ref[:, :] = acc.astype(jnp.float32)

        for h in range(N_DEV - 1):
            send_c = jnp.mod(my - h, N_DEV)
            recv_c = jnp.mod(my - h - 1, N_DEV)
            if h > 0:
                pl.semaphore_wait(credit_sem, 1)
            rdma = pltpu.make_async_remote_copy(
                src_ref=out_ref.at[pl.ds(send_c * CHUNK, CHUNK), :],
                dst_ref=comm_ref,
                send_sem=send_sem,
                recv_sem=recv_sem,
                device_id=(right,),
                device_id_type=pl.DeviceIdType.MESH,
            )
            rdma.start()
            rdma.wait_send()
            rdma.wait_recv()
            rows = pl.ds(recv_c * CHUNK, CHUNK)
            out_ref[rows, :] = out_ref[rows, :] + comm_ref[:, :]
            pl.semaphore_signal(
                credit_sem, inc=1,
                device_id=(left,), device_id_type=pl.DeviceIdType.MESH,
            )

        for g in range(N_DEV - 1):
            send_c = jnp.mod(my + 1 - g, N_DEV)
            rows_s = pl.ds(send_c * CHUNK, CHUNK)
            pl.semaphore_wait(credit_sem, 1)
            rdma = pltpu.make_async_remote_copy(
                src_ref=out_ref.at[rows_s, :],
                dst_ref=out_ref.at[rows_s, :],
                send_sem=send_sem,
                recv_sem=recv_sem,
                device_id=(right,),
                device_id_type=pl.DeviceIdType.MESH,
            )
            rdma.start()
            rdma.wait_send()
            rdma.wait_recv()
            pl.semaphore_signal(
                credit_sem, inc=1,
                device_id=(left,), device_id_type=pl.DeviceIdType.MESH,
            )
        pl.semaphore_wait(credit_sem, 1)

        out_ref[:, :] = out_ref[:, :] * (sx_ref[0] * sw_ref[0])

    return pl.pallas_call(
        body,
        out_shape=jax.ShapeDtypeStruct((M, N), jnp.float32),
        in_specs=[
            pl.BlockSpec(memory_space=pltpu.VMEM),
            pl.BlockSpec(memory_space=pltpu.VMEM),
            pl.BlockSpec(memory_space=pltpu.SMEM),
            pl.BlockSpec(memory_space=pltpu.SMEM),
        ],
        out_specs=pl.BlockSpec(memory_space=pltpu.VMEM),
        scratch_shapes=[
            pltpu.VMEM((CHUNK, N), jnp.float32),
            pltpu.SemaphoreType.DMA,
            pltpu.SemaphoreType.DMA,
            pltpu.SemaphoreType.REGULAR,
        ],
        compiler_params=pltpu.CompilerParams(collective_id=0),
    )(x, w_mat, scale_x, scale_w)


# baseline (device time: 1176464 ns/iter reference)
import jax
import jax.numpy as jnp
from jax import lax
from jax.experimental import pallas as pl
from jax.experimental.pallas import tpu as pltpu

N_DEV = 32
M = 4096
N = 2048
CHUNK = M // N_DEV


def kernel(x, w_mat, scale_x, scale_w):
    def body(x_ref, w_ref, sx_ref, sw_ref, out_ref,
             comm_ref, send_sem, recv_sem, credit_sem):
        my = lax.axis_index("i")
        left = jnp.mod(my - 1, N_DEV)
        right = jnp.mod(my + 1, N_DEV)

        barrier = pltpu.get_barrier_semaphore()
        for nbr in (left, right):
            pl.semaphore_signal(
                barrier, inc=1,
                device_id=(nbr,), device_id_type=pl.DeviceIdType.MESH,
            )
        pl.semaphore_wait(barrier, 2)

        for c in range(N_DEV):
            rows = pl.ds(c * CHUNK, CHUNK)
            acc = lax.dot_general(
                x_ref[rows, :], w_ref[:, :],
                dimension_numbers=(((1,), (0,)), ((), ())),
                preferred_element_type=jnp.int32,
            )
            out_ref[rows, :] = acc.astype(jnp.float32)

        for h in range(N_DEV - 1):
            send_c = jnp.mod(my - h, N_DEV)
            recv_c = jnp.mod(my - h - 1, N_DEV)
            if h > 0:
                pl.semaphore_wait(credit_sem, 1)
            rdma = pltpu.make_async_remote_copy(
                src_ref=out_ref.at[pl.ds(send_c * CHUNK, CHUNK), :],
                dst_ref=comm_ref,
                send_sem=send_sem,
                recv_sem=recv_sem,
                device_id=(right,),
                device_id_type=pl.DeviceIdType.MESH,
            )
            rdma.start()
            rdma.wait_send()
            rdma.wait_recv()
            rows = pl.ds(recv_c * CHUNK, CHUNK)
            out_ref[rows, :] = out_ref[rows, :] + comm_ref[:, :]
            pl.semaphore_signal(
                credit_sem, inc=1,
                device_id=(left,), device_id_type=pl.DeviceIdType.MESH,
            )

        for g in range(N_DEV - 1):
            send_c = jnp.mod(my + 1 - g, N_DEV)
            rows_s = pl.ds(send_c * CHUNK, CHUNK)
            pl.semaphore_wait(credit_sem, 1)
            rdma = pltpu.make_async_remote_copy(
                src_ref=out_ref.at[rows_s, :],
                dst_ref=out_ref.at[rows_s, :],
                send_sem=send_sem,
                recv_sem=recv_sem,
                device_id=(right,),
                device_id_type=pl.DeviceIdType.MESH,
            )
            rdma.start()
            rdma.wait_send()
            rdma.wait_recv()
            pl.semaphore_signal(
                credit_sem, inc=1,
                device_id=(left,), device_id_type=pl.DeviceIdType.MESH,
            )
        pl.semaphore_wait(credit_sem, 1)

        out_ref[:, :] = out_ref[:, :] * (sx_ref[0] * sw_ref[0])

    return pl.pallas_call(
        body,
        out_shape=jax.ShapeDtypeStruct((M, N), jnp.float32),
        in_specs=[
            pl.BlockSpec(memory_space=pltpu.VMEM),
            pl.BlockSpec(memory_space=pltpu.VMEM),
            pl.BlockSpec(memory_space=pltpu.SMEM),
            pl.BlockSpec(memory_space=pltpu.SMEM),
        ],
        out_specs=pl.BlockSpec(memory_space=pltpu.VMEM),
        scratch_shapes=[
            pltpu.VMEM((CHUNK, N), jnp.float32),
            pltpu.SemaphoreType.DMA,
            pltpu.SemaphoreType.DMA,
            pltpu.SemaphoreType.REGULAR,
        ],
        compiler_params=pltpu.CompilerParams(
            collective_id=0,
            vmem_limit_bytes=100 * 1024 * 1024,
        ),
    )(x, w_mat, scale_x, scale_w)


# device time: 467282 ns/iter; 2.5177x vs baseline; 2.5177x over previous
import jax
import jax.numpy as jnp
from jax import lax
from jax.experimental import pallas as pl
from jax.experimental.pallas import tpu as pltpu

N_DEV = 32
N_RING = 16
M = 4096
N = 2048
CHUNK = M // N_DEV
MESH = pl.DeviceIdType.MESH


def _ring_pos_to_id(q, x):
    q = jnp.mod(q, N_RING)
    zq = q // 4
    yq = jnp.where(zq % 2 == 0, q % 4, 3 - (q % 4))
    return zq * 8 + yq * 2 + jnp.where(yq % 2 == 0, x, 1 - x)


def kernel(x, w_mat, scale_x, scale_w):
    def body(x_ref, w_ref, sx_ref, sw_ref, out_ref,
             comm_ref, x_comm_ref,
             r_send_sems, r_recv_sems, x_send_sems, x_recv_sems,
             r_credit, x_credit):
        my = lax.axis_index("i")
        z = my // 8
        rr = my % 8
        yy = rr // 2
        xi = rr % 2
        xc = jnp.where(yy % 2 == 0, xi, 1 - xi)
        q = z * 4 + jnp.where(z % 2 == 0, yy, 3 - yy)

        partner = jnp.bitwise_xor(my, 1)
        ring_right = _ring_pos_to_id(q + 1, xc)
        ring_left = _ring_pos_to_id(q - 1, xc)

        def blk(i):
            return xc * N_RING + jnp.mod(i, N_RING)

        def pblk(i):
            return (1 - xc) * N_RING + jnp.mod(i, N_RING)

        def rows(b):
            return pl.ds(b * CHUNK, CHUNK)

        barrier = pltpu.get_barrier_semaphore()
        for nbr in (partner, ring_left, ring_right):
            pl.semaphore_signal(barrier, inc=1, device_id=(nbr,),
                                device_id_type=MESH)
        pl.semaphore_wait(barrier, 3)

        for c in range(N_DEV):
            r_ = pl.ds(c * CHUNK, CHUNK)
            acc = lax.dot_general(
                x_ref[r_, :], w_ref[:, :],
                dimension_numbers=(((1,), (0,)), ((), ())),
                preferred_element_type=jnp.int32,
            )
            out_ref[r_, :] = acc.astype(jnp.float32)

        x_descs = {}

        def x_start(k):
            if k >= 2:
                x_descs[k - 2].wait_send()
                pl.semaphore_wait(x_credit, 1)
            if k < N_RING:
                src = out_ref.at[rows(pblk(q - k)), :]
                dst = x_comm_ref.at[k % 2]
            else:
                b = blk(q + 1 - (k - N_RING))
                src = out_ref.at[rows(b), :]
                dst = out_ref.at[rows(b), :]
            d = pltpu.make_async_remote_copy(
                src_ref=src, dst_ref=dst,
                send_sem=x_send_sems.at[k % 2],
                recv_sem=x_recv_sems.at[k % 2],
                device_id=(partner,), device_id_type=MESH,
            )
            d.start()
            x_descs[k] = d

        def x_finish(k):
            x_descs[k].wait_recv()
            if k < N_RING:
                r_ = rows(blk(q - k))
                out_ref[r_, :] = out_ref[r_, :] + x_comm_ref[k % 2, :, :]
            pl.semaphore_signal(x_credit, inc=1, device_id=(partner,),
                                device_id_type=MESH)

        r_descs = {}

        def ring_step(t):
            if t >= 2:
                r_descs[t - 2].wait_send()
                pl.semaphore_wait(r_credit, 1)
            if t < N_RING - 1:
                src = out_ref.at[rows(blk(q - t)), :]
                dst = comm_ref.at[t % 2]
            else:
                b = blk(q + 1 - (t - (N_RING - 1)))
                src = out_ref.at[rows(b), :]
                dst = out_ref.at[rows(b), :]
            d = pltpu.make_async_remote_copy(
                src_ref=src, dst_ref=dst,
                send_sem=r_send_sems.at[t % 2],
                recv_sem=r_recv_sems.at[t % 2],
                device_id=(ring_right,), device_id_type=MESH,
            )
            d.start()
            r_descs[t] = d
            d.wait_recv()
            if t < N_RING - 1:
                r_ = rows(blk(q - t - 1))
                out_ref[r_, :] = out_ref[r_, :] + comm_ref[t % 2, :, :]
            pl.semaphore_signal(r_credit, inc=1, device_id=(ring_left,),
                                device_id_type=MESH)

        x_start(0)
        x_start(1)
        x_finish(0)
        for t in range(N_RING - 1):
            ring_step(t)
            if t + 2 < N_RING:
                x_start(t + 2)
            x_finish(t + 1)
        for g in range(N_RING - 1):
            x_start(N_RING + g)
            ring_step(N_RING - 1 + g)
            x_finish(N_RING + g)
        x_start(2 * N_RING - 1)
        x_finish(2 * N_RING - 1)

        pl.semaphore_wait(r_credit, 2)
        pl.semaphore_wait(x_credit, 2)
        r_descs[2 * (N_RING - 1) - 2].wait_send()
        r_descs[2 * (N_RING - 1) - 1].wait_send()
        x_descs[2 * N_RING - 2].wait_send()
        x_descs[2 * N_RING - 1].wait_send()

        out_ref[:, :] = out_ref[:, :] * (sx_ref[0] * sw_ref[0])

    return pl.pallas_call(
        body,
        out_shape=jax.ShapeDtypeStruct((M, N), jnp.float32),
        in_specs=[
            pl.BlockSpec(memory_space=pltpu.VMEM),
            pl.BlockSpec(memory_space=pltpu.VMEM),
            pl.BlockSpec(memory_space=pltpu.SMEM),
            pl.BlockSpec(memory_space=pltpu.SMEM),
        ],
        out_specs=pl.BlockSpec(memory_space=pltpu.VMEM),
        scratch_shapes=[
            pltpu.VMEM((2, CHUNK, N), jnp.float32),
            pltpu.VMEM((2, CHUNK, N), jnp.float32),
            pltpu.SemaphoreType.DMA((2,)),
            pltpu.SemaphoreType.DMA((2,)),
            pltpu.SemaphoreType.DMA((2,)),
            pltpu.SemaphoreType.DMA((2,)),
            pltpu.SemaphoreType.REGULAR,
            pltpu.SemaphoreType.REGULAR,
        ],
        compiler_params=pltpu.CompilerParams(
            collective_id=0,
            vmem_limit_bytes=100 * 1024 * 1024,
        ),
    )(x, w_mat, scale_x, scale_w)


# device time: 412226 ns/iter; 2.8539x vs baseline; 1.1336x over previous
import jax
import jax.numpy as jnp
from jax import lax
from jax.experimental import pallas as pl
from jax.experimental.pallas import tpu as pltpu

N_DEV = 32
N_RING = 16
M = 4096
N = 2048
CHUNK = M // N_DEV
HALF = CHUNK // 2
R_STEPS = 2 * (N_RING - 1)
MESH = pl.DeviceIdType.MESH


def _ring_pos_to_id(q, x):
    q = jnp.mod(q, N_RING)
    zq = q // 4
    yq = jnp.where(zq % 2 == 0, q % 4, 3 - (q % 4))
    return zq * 8 + yq * 2 + jnp.where(yq % 2 == 0, x, 1 - x)


def kernel(x, w_mat, scale_x, scale_w):
    def body(x_ref, w_ref, sx_ref, sw_ref, out_ref,
             comm_a_ref, comm_b_ref, x_comm_ref,
             a_send_sems, a_recv_sems, b_send_sems, b_recv_sems,
             x_send_sems, x_recv_sems,
             a_credit, b_credit, x_credit):
        my = lax.axis_index("i")
        z = my // 8
        rr = my % 8
        yy = rr // 2
        xi = rr % 2
        xc = jnp.where(yy % 2 == 0, xi, 1 - xi)
        q = z * 4 + jnp.where(z % 2 == 0, yy, 3 - yy)

        partner = jnp.bitwise_xor(my, 1)
        ring_right = _ring_pos_to_id(q + 1, xc)
        ring_left = _ring_pos_to_id(q - 1, xc)

        scale = sx_ref[0] * sw_ref[0]

        def blk(i):
            return xc * N_RING + jnp.mod(i, N_RING)

        def pblk(i):
            return (1 - xc) * N_RING + jnp.mod(i, N_RING)

        def rows(b):
            return pl.ds(b * CHUNK, CHUNK)

        def rows_h(b, h):
            return pl.ds(b * CHUNK + h * HALF, HALF)

        for c in range(N_DEV):
            r_ = pl.ds(c * CHUNK, CHUNK)
            acc = lax.dot_general(
                x_ref[r_, :], w_ref[:, :],
                dimension_numbers=(((1,), (0,)), ((), ())),
                preferred_element_type=jnp.int32,
            )
            out_ref[r_, :] = acc.astype(jnp.float32)

        barrier = pltpu.get_barrier_semaphore()
        for nbr in (partner, ring_left, ring_right):
            pl.semaphore_signal(barrier, inc=1, device_id=(nbr,),
                                device_id_type=MESH)
        pl.semaphore_wait(barrier, 3)

        x_descs = {}

        def x_start(k):
            if k >= 2:
                x_descs[k - 2].wait_send()
                pl.semaphore_wait(x_credit, 1)
            if k < N_RING:
                src = out_ref.at[rows(pblk(q - k)), :]
                dst = x_comm_ref.at[k % 2]
            else:
                b = blk(q + 1 - (k - N_RING))
                src = out_ref.at[rows(b), :]
                dst = out_ref.at[rows(b), :]
            d = pltpu.make_async_remote_copy(
                src_ref=src, dst_ref=dst,
                send_sem=x_send_sems.at[k % 2],
                recv_sem=x_recv_sems.at[k % 2],
                device_id=(partner,), device_id_type=MESH,
            )
            d.start()
            x_descs[k] = d

        def x_finish(k):
            x_descs[k].wait_recv()
            if k < N_RING:
                r_ = rows(blk(q - k))
                out_ref[r_, :] = out_ref[r_, :] + x_comm_ref[k % 2, :, :]
            pl.semaphore_signal(x_credit, inc=1, device_id=(partner,),
                                device_id_type=MESH)

        a_descs = {}
        b_descs = {}

        def _r_start(t, h, descs, comm, send_sems, recv_sems, credit):
            if t >= 2:
                descs[t - 2].wait_send()
                pl.semaphore_wait(credit, 1)
            if t < N_RING - 1:
                src = out_ref.at[rows_h(blk(q - t), h), :]
                dst = comm.at[t % 2]
            else:
                b = blk(q + 1 - (t - (N_RING - 1)))
                src = out_ref.at[rows_h(b, h), :]
                dst = out_ref.at[rows_h(b, h), :]
            d = pltpu.make_async_remote_copy(
                src_ref=src, dst_ref=dst,
                send_sem=send_sems.at[t % 2],
                recv_sem=recv_sems.at[t % 2],
                device_id=(ring_right,), device_id_type=MESH,
            )
            d.start()
            descs[t] = d

        def _r_finish(t, h, descs, comm, credit):
            descs[t].wait_recv()
            if t < N_RING - 1:
                r_ = rows_h(blk(q - t - 1), h)
                out_ref[r_, :] = out_ref[r_, :] + comm[t % 2, :, :]
            pl.semaphore_signal(credit, inc=1, device_id=(ring_left,),
                                device_id_type=MESH)

        def ra_start(t):
            _r_start(t, 0, a_descs, comm_a_ref, a_send_sems, a_recv_sems,
                     a_credit)

        def ra_finish(t):
            _r_finish(t, 0, a_descs, comm_a_ref, a_credit)

        def rb_start(t):
            _r_start(t, 1, b_descs, comm_b_ref, b_send_sems, b_recv_sems,
                     b_credit)

        def rb_finish(t):
            _r_finish(t, 1, b_descs, comm_b_ref, b_credit)

        x_start(0)
        x_start(1)
        x_finish(0)
        ra_start(0)
        rb_start(0)
        for t in range(R_STEPS):
            ra_finish(t)
            j = t + 1
            if j <= N_RING - 1:
                if j + 1 <= N_RING - 1:
                    x_start(j + 1)
                x_finish(j)
            if t == N_RING - 2:
                r_ = rows_h(blk(q + 1), 0)
                out_ref[r_, :] = out_ref[r_, :] * scale
            if t < R_STEPS - 1:
                ra_start(t + 1)
            rb_finish(t)
            if t == N_RING - 2:
                r_ = rows_h(blk(q + 1), 1)
                out_ref[r_, :] = out_ref[r_, :] * scale
            if t < R_STEPS - 1:
                rb_start(t + 1)
            if t >= N_RING - 2:
                x_start(t + 2)
            if t >= N_RING - 1:
                x_finish(t + 1)
        x_finish(2 * N_RING - 1)

        pl.semaphore_wait(a_credit, 2)
        pl.semaphore_wait(b_credit, 2)
        pl.semaphore_wait(x_credit, 2)
        a_descs[R_STEPS - 2].wait_send()
        a_descs[R_STEPS - 1].wait_send()
        b_descs[R_STEPS - 2].wait_send()
        b_descs[R_STEPS - 1].wait_send()
        x_descs[2 * N_RING - 2].wait_send()
        x_descs[2 * N_RING - 1].wait_send()

    return pl.pallas_call(
        body,
        out_shape=jax.ShapeDtypeStruct((M, N), jnp.float32),
        in_specs=[
            pl.BlockSpec(memory_space=pltpu.VMEM),
            pl.BlockSpec(memory_space=pltpu.VMEM),
            pl.BlockSpec(memory_space=pltpu.SMEM),
            pl.BlockSpec(memory_space=pltpu.SMEM),
        ],
        out_specs=pl.BlockSpec(memory_space=pltpu.VMEM),
        scratch_shapes=[
            pltpu.VMEM((2, HALF, N), jnp.float32),
            pltpu.VMEM((2, HALF, N), jnp.float32),
            pltpu.VMEM((2, CHUNK, N), jnp.float32),
            pltpu.SemaphoreType.DMA((2,)),
            pltpu.SemaphoreType.DMA((2,)),
            pltpu.SemaphoreType.DMA((2,)),
            pltpu.SemaphoreType.DMA((2,)),
            pltpu.SemaphoreType.DMA((2,)),
            pltpu.SemaphoreType.DMA((2,)),
            pltpu.SemaphoreType.REGULAR,
            pltpu.SemaphoreType.REGULAR,
            pltpu.SemaphoreType.REGULAR,
        ],
        compiler_params=pltpu.CompilerParams(
            collective_id=0,
            vmem_limit_bytes=100 * 1024 * 1024,
        ),
    )(x, w_mat, scale_x, scale_w)
